# baseline (speedup 1.0000x reference)
import functools

import jax
import jax.numpy as jnp
from jax import lax
from jax.experimental import pallas as pl
from jax.experimental.pallas import tpu as pltpu
from jax.experimental.pallas import tpu_sc as plsc

D = 32
DP = D // 2
L = 16
NW = 32
CH = 1280


def _dot_sigmoid_chunk(rows_a, rows_b, outbuf):
    pa = rows_a
    pb = rows_b

    def group(g, carry):
        row_ids = g * L + lax.iota(jnp.int32, L)
        lane = lax.iota(jnp.int32, L)
        accs = [jnp.zeros((L,), jnp.float32) for _ in range(4)]
        for dp in range(DP):
            col = (lane + dp) & (DP - 1)
            ga = plsc.load_gather(pa, [row_ids, col])
            gb = plsc.load_gather(pb, [row_ids, col])
            a0, a1 = plsc.unpack(plsc.bitcast(ga, jnp.bfloat16),
                                 format=plsc.PackFormat.INTERLEAVED)
            b0, b1 = plsc.unpack(plsc.bitcast(gb, jnp.bfloat16),
                                 format=plsc.PackFormat.INTERLEAVED)
            k = (dp & 1) * 2
            accs[k] = accs[k] + a0 * b0
            accs[k + 1] = accs[k + 1] + a1 * b1
        acc = (accs[0] + accs[1]) + (accs[2] + accs[3])
        outbuf[pl.ds(g * L, L)] = 1.0 / (1.0 + jnp.exp(-acc))
        return carry

    lax.fori_loop(0, CH // L, group, None)


def kernel(B, V, W, user_idx, item_idx, user_idx2, trust_idx2):
    n_rating = user_idx.shape[0]
    n_trust = user_idx2.shape[0]
    assert n_rating % (NW * 2 * CH) == 0 and n_trust % (NW * 2 * CH) == 0

    def to_packed(t):
        bits = lax.bitcast_convert_type(t, jnp.uint32)
        even = bits[:, 0::2] >> 16
        odd = bits[:, 1::2] & jnp.uint32(0xFFFF0000)
        return lax.bitcast_convert_type(even | odd, jnp.int32)

    Bh = to_packed(B)
    Vh = to_packed(V)
    Wh = to_packed(W)

    mesh = plsc.VectorSubcoreMesh(core_axis_name="c", subcore_axis_name="s")

    @functools.partial(
        pl.kernel,
        out_type=(
            jax.ShapeDtypeStruct((n_rating,), jnp.float32),
            jax.ShapeDtypeStruct((n_trust,), jnp.float32),
        ),
        mesh=mesh,
        compiler_params=pltpu.CompilerParams(
            needs_layout_passes=False, use_tc_tiling_on_sc=False),
        scratch_types=[
            pltpu.VMEM((CH,), jnp.int32),
            pltpu.VMEM((CH,), jnp.int32),
            pltpu.VMEM((CH,), jnp.int32),
            pltpu.VMEM((CH,), jnp.int32),
            pltpu.VMEM((CH, DP), jnp.int32),
            pltpu.VMEM((CH, DP), jnp.int32),
            pltpu.VMEM((CH, DP), jnp.int32),
            pltpu.VMEM((CH, DP), jnp.int32),
            pltpu.VMEM((CH,), jnp.float32),
            pltpu.SemaphoreType.DMA,
            pltpu.SemaphoreType.DMA,
            pltpu.SemaphoreType.DMA,
            pltpu.SemaphoreType.DMA,
        ],
    )
    def run(B_h, V_h, W_h, ui_h, ii_h, ui2_h, ti2_h, outr_h, outt_h,
            ia0, ia1, ib0, ib1, ra0, ra1, rb0, rb1, outbuf,
            sa0, sa1, sb0, sb1):
        wid = lax.axis_index("s") * 2 + lax.axis_index("c")
        idx_a = (ia0, ia1)
        idx_b = (ib0, ib1)
        rows_a = (ra0, ra1)
        rows_b = (rb0, rb1)
        sem_a = (sa0, sa1)
        sem_b = (sb0, sb1)

        def phase(tab_a_h, tab_b_h, ia_h, ib_h, out_h, n):
            per_w = n // NW
            nch = per_w // CH
            base_w = wid * per_w

            def issue(c, k):
                base = base_w + c * CH
                pltpu.sync_copy(ia_h.at[pl.ds(base, CH)], idx_a[k])
                pltpu.sync_copy(ib_h.at[pl.ds(base, CH)], idx_b[k])
                pltpu.async_copy(tab_a_h.at[idx_a[k]], rows_a[k], sem_a[k])
                pltpu.async_copy(tab_b_h.at[idx_b[k]], rows_b[k], sem_b[k])

            def drain(k):
                pltpu.make_async_copy(
                    tab_a_h.at[idx_a[k]], rows_a[k], sem_a[k]).wait()
                pltpu.make_async_copy(
                    tab_b_h.at[idx_b[k]], rows_b[k], sem_b[k]).wait()

            def finish(c, k):
                drain(k)
                _dot_sigmoid_chunk(rows_a[k], rows_b[k], outbuf)
                pltpu.sync_copy(outbuf, out_h.at[pl.ds(base_w + c * CH, CH)])

            issue(0, 0)

            def pair(p, carry):
                c0 = p * 2
                issue(c0 + 1, 1)
                finish(c0, 0)
                @pl.when(c0 + 2 < nch)
                def _():
                    issue(c0 + 2, 0)
                finish(c0 + 1, 1)
                return carry

            lax.fori_loop(0, nch // 2, pair, None)

        phase(B_h, V_h, ui_h, ii_h, outr_h, n_rating)
        phase(B_h, W_h, ui2_h, ti2_h, outt_h, n_trust)

    return run(Bh, Vh, Wh, user_idx, item_idx, user_idx2, trust_idx2)


# speedup vs baseline: 16.2576x; 16.2576x over previous
import functools

import jax
import jax.numpy as jnp
from jax import lax
from jax.experimental import pallas as pl
from jax.experimental.pallas import tpu as pltpu
from jax.experimental.pallas import tpu_sc as plsc

D = 32
L = 16
NW = 32
CH = 640


def _dot_sigmoid_chunk(rows_a, rows_b, outbuf):
    def group(g, carry):
        row_ids = g * L + lax.iota(jnp.int32, L)
        lane = lax.iota(jnp.int32, L)
        accs = [jnp.zeros((L,), jnp.float32) for _ in range(4)]
        for d in range(D):
            col = (lane + d) & (D - 1)
            a = plsc.load_gather(rows_a, [row_ids, col])
            b = plsc.load_gather(rows_b, [row_ids, col])
            accs[d % 4] = accs[d % 4] + a * b
        acc = (accs[0] + accs[1]) + (accs[2] + accs[3])
        outbuf[pl.ds(g * L, L)] = 1.0 / (1.0 + jnp.exp(-acc))
        return carry

    lax.fori_loop(0, CH // L, group, None)


def _make_pass(n):
    assert n % (NW * 2 * CH) == 0
    mesh = plsc.VectorSubcoreMesh(core_axis_name="c", subcore_axis_name="s")

    @functools.partial(
        pl.kernel,
        out_type=jax.ShapeDtypeStruct((n,), jnp.float32),
        mesh=mesh,
        compiler_params=pltpu.CompilerParams(
            needs_layout_passes=False, use_tc_tiling_on_sc=False),
        scratch_types=[
            pltpu.VMEM((CH,), jnp.int32),
            pltpu.VMEM((CH,), jnp.int32),
            pltpu.VMEM((CH,), jnp.int32),
            pltpu.VMEM((CH,), jnp.int32),
            pltpu.VMEM((CH, D), jnp.float32),
            pltpu.VMEM((CH, D), jnp.float32),
            pltpu.VMEM((CH, D), jnp.float32),
            pltpu.VMEM((CH, D), jnp.float32),
            pltpu.VMEM((CH,), jnp.float32),
            pltpu.SemaphoreType.DMA,
            pltpu.SemaphoreType.DMA,
            pltpu.SemaphoreType.DMA,
            pltpu.SemaphoreType.DMA,
        ],
    )
    def run(tab_a_h, tab_b_h, ia_h, ib_h, out_h,
            ia0, ia1, ib0, ib1, ra0, ra1, rb0, rb1, outbuf,
            sa0, sa1, sb0, sb1):
        wid = lax.axis_index("s") * 2 + lax.axis_index("c")
        idx_a = (ia0, ia1)
        idx_b = (ib0, ib1)
        rows_a = (ra0, ra1)
        rows_b = (rb0, rb1)
        sem_a = (sa0, sa1)
        sem_b = (sb0, sb1)

        per_w = n // NW
        nch = per_w // CH
        base_w = wid * per_w

        def issue(c, k):
            base = base_w + c * CH
            pltpu.sync_copy(ia_h.at[pl.ds(base, CH)], idx_a[k])
            pltpu.sync_copy(ib_h.at[pl.ds(base, CH)], idx_b[k])
            pltpu.async_copy(tab_a_h.at[idx_a[k]], rows_a[k], sem_a[k])
            pltpu.async_copy(tab_b_h.at[idx_b[k]], rows_b[k], sem_b[k])

        def drain(k):
            pltpu.make_async_copy(
                tab_a_h.at[idx_a[k]], rows_a[k], sem_a[k]).wait()
            pltpu.make_async_copy(
                tab_b_h.at[idx_b[k]], rows_b[k], sem_b[k]).wait()

        def finish(c, k):
            drain(k)
            _dot_sigmoid_chunk(rows_a[k], rows_b[k], outbuf)
            pltpu.sync_copy(outbuf, out_h.at[pl.ds(base_w + c * CH, CH)])

        issue(0, 0)

        def pair(p, carry):
            c0 = p * 2
            issue(c0 + 1, 1)
            finish(c0, 0)
            @pl.when(c0 + 2 < nch)
            def _():
                issue(c0 + 2, 0)
            finish(c0 + 1, 1)
            return carry

        lax.fori_loop(0, nch // 2, pair, None)

    return run


def kernel(B, V, W, user_idx, item_idx, user_idx2, trust_idx2):
    pred_r = _make_pass(user_idx.shape[0])(B, V, user_idx, item_idx)
    pred_t = _make_pass(user_idx2.shape[0])(B, W, user_idx2, trust_idx2)
    return (pred_r, pred_t)
